# Initial kernel scaffold; baseline (speedup 1.0000x reference)
import jax
import jax.numpy as jnp
from jax.experimental import pallas as pl


def kernel(agent_observations, W1, b1, W2, b2, Wo, bo):
    raise NotImplementedError("write your pallas kernel here")


# speedup vs baseline: 19.0715x; 19.0715x over previous
import jax
import jax.numpy as jnp
from jax import lax
from jax.experimental import pallas as pl

_B, _A, _D, _H, _OUT, _K = 50, 200, 128, 128, 32, 32


def _env_kernel(obs_ref, W1_ref, b1_ref, W2_ref, b2_ref, Wo_ref, bo_ref, out_ref):
    x = obs_ref[0]

    pos = x[:, 0:2]
    ii = lax.broadcasted_iota(jnp.int32, (_A, _A), 0)
    jj = lax.broadcasted_iota(jnp.int32, (_A, _A), 1)
    eyeA = (ii == jj).astype(jnp.float32)
    posT = lax.dot_general(pos, eyeA, (((0,), (0,)), ((), ())),
                           preferred_element_type=jnp.float32, precision=lax.Precision.HIGHEST)
    dx = pos[:, 0:1] - posT[0:1, :]
    dy = pos[:, 1:2] - posT[1:2, :]
    d = jnp.sqrt(dx * dx + dy * dy)

    bits = lax.bitcast_convert_type(d, jnp.int32)
    kp1 = _K + 1
    lo0 = jnp.zeros((_A, 1), jnp.int32)
    hi0 = jnp.full((_A, 1), 0x7F800000, jnp.int32)

    def bs_body(_, carry):
        lo, hi = carry
        mid = lo + lax.div(hi - lo, 2)
        cnt = jnp.sum((bits <= mid).astype(jnp.int32), axis=1, keepdims=True)
        ge = cnt >= kp1
        return jnp.where(ge, lo, mid + 1), jnp.where(ge, mid, hi)

    _, t = lax.fori_loop(0, 31, bs_body, (lo0, hi0))

    lt_f = (bits < t).astype(jnp.float32)
    n_lt = jnp.sum(lt_f, axis=1, keepdims=True)
    eq_f = (bits == t).astype(jnp.float32)
    mn = jnp.min(bits, axis=1, keepdims=True)
    eqm_f = (bits == mn).astype(jnp.float32)

    triu = (ii <= jj).astype(jnp.float32)
    stacked = jnp.concatenate([eq_f, eqm_f], axis=0)
    pref = lax.dot_general(stacked, triu, (((1,), (0,)), ((), ())),
                           preferred_element_type=jnp.float32, precision=lax.Precision.HIGHEST)
    pref_eq = pref[0:_A, :]
    pref_mn = pref[_A:, :]

    r = kp1 - n_lt
    sel = lt_f + eq_f * (pref_eq <= r).astype(jnp.float32)
    first_min = eqm_f * (pref_mn == 1.0).astype(jnp.float32)
    M = sel - first_min

    ones_col = jnp.ones((_A, 1), jnp.float32)
    deg = lax.dot_general(M, ones_col, (((0,), (0,)), ((), ())),
                          preferred_element_type=jnp.float32, precision=lax.Precision.HIGHEST) + 2.0
    dinv = 1.0 / jnp.sqrt(deg)

    def conv(xin, W, b):
        xt = lax.dot_general(xin, W, (((1,), (1,)), ((), ())),
                             preferred_element_type=jnp.float32, precision=lax.Precision.HIGHEST)
        u = dinv * xt
        z = lax.dot_general(M, u, (((0,), (0,)), ((), ())),
                            preferred_element_type=jnp.float32, precision=lax.Precision.HIGHEST)
        return dinv * z + 2.0 * dinv * u + b

    h1 = jnp.tanh(conv(x, W1_ref[...], b1_ref[...]))
    h2 = jnp.tanh(conv(h1, W2_ref[...], b2_ref[...]))
    out = lax.dot_general(h2, Wo_ref[...], (((1,), (1,)), ((), ())),
                          preferred_element_type=jnp.float32, precision=lax.Precision.HIGHEST) + bo_ref[...]
    out_ref[0] = out


def _build(interpret=False):
    return pl.pallas_call(
        _env_kernel,
        grid=(_B,),
        in_specs=[
            pl.BlockSpec((1, _A, _D), lambda b: (b, 0, 0)),
            pl.BlockSpec((_H, _D), lambda b: (0, 0)),
            pl.BlockSpec((1, _H), lambda b: (0, 0)),
            pl.BlockSpec((_H, _H), lambda b: (0, 0)),
            pl.BlockSpec((1, _H), lambda b: (0, 0)),
            pl.BlockSpec((_OUT, _H), lambda b: (0, 0)),
            pl.BlockSpec((1, _OUT), lambda b: (0, 0)),
        ],
        out_specs=pl.BlockSpec((1, _A, _OUT), lambda b: (b, 0, 0)),
        out_shape=jax.ShapeDtypeStruct((_B, _A, _OUT), jnp.float32),
        interpret=interpret,
    )


@jax.jit
def kernel(agent_observations, W1, b1, W2, b2, Wo, bo):
    obs = agent_observations.astype(jnp.float32)
    return _build()(obs, W1, b1.reshape(1, _H), W2, b2.reshape(1, _H),
                    Wo, bo.reshape(1, _OUT))


# speedup vs baseline: 37.3966x; 1.9609x over previous
import jax
import jax.numpy as jnp
from jax import lax
from jax.experimental import pallas as pl

_B, _A, _D, _H, _OUT, _K = 50, 200, 128, 128, 32, 32
_C = 10
_N = _C * _A

_HI = lax.Precision.HIGHEST


def _env_kernel(obs_ref, W1_ref, b1_ref, W2_ref, b2_ref, Wo_ref, bo_ref, out_ref):
    obs = obs_ref[...]
    X = obs.reshape(_N, _D)

    px_row = obs[:, :, 0]
    py_row = obs[:, :, 1]
    gi = lax.broadcasted_iota(jnp.int32, (_N, _C), 0)
    ei = lax.broadcasted_iota(jnp.int32, (_N, _C), 1)
    E = (gi // _A == ei).astype(jnp.float32)
    PRX = lax.dot_general(E, px_row, (((1,), (0,)), ((), ())),
                          preferred_element_type=jnp.float32, precision=_HI)
    PRY = lax.dot_general(E, py_row, (((1,), (0,)), ((), ())),
                          preferred_element_type=jnp.float32, precision=_HI)
    dx = X[:, 0:1] - PRX
    dy = X[:, 1:2] - PRY
    d = jnp.sqrt(dx * dx + dy * dy)

    bits = lax.bitcast_convert_type(d, jnp.int32)
    kp1 = float(_K + 1)
    ones_colA = jnp.ones((_A, 1), jnp.float32)
    lo0 = jnp.zeros((_N, 1), jnp.int32)
    hi0 = jnp.full((_N, 1), 0x7F800000, jnp.int32)

    def bs_body(_, carry):
        lo, hi = carry
        mid = lo + lax.div(hi - lo, 2)
        le_f = (bits <= mid).astype(jnp.float32)
        cnt = lax.dot_general(le_f, ones_colA, (((1,), (0,)), ((), ())),
                              preferred_element_type=jnp.float32)
        ge = cnt >= kp1
        return jnp.where(ge, lo, mid + 1), jnp.where(ge, mid, hi)

    _, t = lax.fori_loop(0, 31, bs_body, (lo0, hi0))

    lt_f = (bits < t).astype(jnp.float32)
    eq_f = (bits == t).astype(jnp.float32)
    mn = jnp.min(bits, axis=1, keepdims=True)
    eqm_f = (bits == mn).astype(jnp.float32)
    n_lt = lax.dot_general(lt_f, ones_colA, (((1,), (0,)), ((), ())),
                           preferred_element_type=jnp.float32)

    ii = lax.broadcasted_iota(jnp.int32, (_A, _A), 0)
    jj = lax.broadcasted_iota(jnp.int32, (_A, _A), 1)
    triu = (ii <= jj).astype(jnp.float32)
    stacked = jnp.concatenate([eq_f, eqm_f], axis=0)
    pref = lax.dot_general(stacked, triu, (((1,), (0,)), ((), ())),
                           preferred_element_type=jnp.float32)
    pref_eq = pref[0:_N, :]
    pref_mn = pref[_N:, :]

    r = kp1 - n_lt
    sel = lt_f + eq_f * (pref_eq <= r).astype(jnp.float32)
    first_min = eqm_f * (pref_mn == 1.0).astype(jnp.float32)
    M = sel - first_min

    dinv_parts = []
    for e in range(_C):
        Me = M[e * _A:(e + 1) * _A, :]
        deg_e = lax.dot_general(Me, ones_colA, (((0,), (0,)), ((), ())),
                                preferred_element_type=jnp.float32) + 2.0
        dinv_parts.append(1.0 / jnp.sqrt(deg_e))
    dinv = jnp.concatenate(dinv_parts, axis=0)

    def conv(xin, W, b):
        xt = lax.dot_general(xin, W, (((1,), (1,)), ((), ())),
                             preferred_element_type=jnp.float32,
                             precision=_HI)
        u = dinv * xt
        zs = []
        for e in range(_C):
            Me = M[e * _A:(e + 1) * _A, :]
            ue = u[e * _A:(e + 1) * _A, :]
            zs.append(lax.dot_general(Me, ue, (((0,), (0,)), ((), ())),
                                      preferred_element_type=jnp.float32,
                                      precision=_HI))
        z = jnp.concatenate(zs, axis=0)
        return dinv * z + 2.0 * dinv * u + b

    h1 = jnp.tanh(conv(X, W1_ref[...], b1_ref[...]))
    h2 = jnp.tanh(conv(h1, W2_ref[...], b2_ref[...]))
    out = lax.dot_general(h2, Wo_ref[...], (((1,), (1,)), ((), ())),
                          preferred_element_type=jnp.float32,
                          precision=_HI) + bo_ref[...]
    out_ref[...] = out.reshape(_C, _A, _OUT)


def _build(interpret=False):
    return pl.pallas_call(
        _env_kernel,
        grid=(_B // _C,),
        in_specs=[
            pl.BlockSpec((_C, _A, _D), lambda b: (b, 0, 0)),
            pl.BlockSpec((_H, _D), lambda b: (0, 0)),
            pl.BlockSpec((1, _H), lambda b: (0, 0)),
            pl.BlockSpec((_H, _H), lambda b: (0, 0)),
            pl.BlockSpec((1, _H), lambda b: (0, 0)),
            pl.BlockSpec((_OUT, _H), lambda b: (0, 0)),
            pl.BlockSpec((1, _OUT), lambda b: (0, 0)),
        ],
        out_specs=pl.BlockSpec((_C, _A, _OUT), lambda b: (b, 0, 0)),
        out_shape=jax.ShapeDtypeStruct((_B, _A, _OUT), jnp.float32),
        interpret=interpret,
    )


@jax.jit
def kernel(agent_observations, W1, b1, W2, b2, Wo, bo):
    obs = agent_observations.astype(jnp.float32)
    return _build()(obs, W1, b1.reshape(1, _H), W2, b2.reshape(1, _H),
                    Wo, bo.reshape(1, _OUT))


# speedup vs baseline: 50.6706x; 1.3550x over previous
import jax
import jax.numpy as jnp
from jax import lax
from jax.experimental import pallas as pl

_B, _A, _D, _H, _OUT, _K = 50, 200, 128, 128, 32, 32
_C = 10
_N = _C * _A

_HI = lax.Precision.HIGHEST
_H3 = lax.Precision.DEFAULT


def _env_kernel(obs_ref, W1_ref, b1_ref, W2_ref, b2_ref, Wo_ref, bo_ref, out_ref):
    obs = obs_ref[...]
    X = obs.reshape(_N, _D)

    px_row = obs[:, :, 0]
    py_row = obs[:, :, 1]
    PRX = jnp.repeat(px_row, _A, axis=0)
    PRY = jnp.repeat(py_row, _A, axis=0)
    dx = X[:, 0:1] - PRX
    dy = X[:, 1:2] - PRY
    d = jnp.sqrt(dx * dx + dy * dy)

    bits = lax.bitcast_convert_type(d, jnp.int32)
    kp1 = float(_K + 1)
    ones_colA = jnp.ones((_A, 1), jnp.bfloat16)
    lo0 = jnp.zeros((_N, 1), jnp.int32)
    hi0 = jnp.full((_N, 1), 0x7F800000, jnp.int32)

    def bs_body(_, carry):
        lo, hi = carry
        mid = lo + lax.div(hi - lo, 2)
        le = (bits <= mid).astype(jnp.bfloat16)
        cnt = lax.dot_general(le, ones_colA, (((1,), (0,)), ((), ())),
                              preferred_element_type=jnp.float32)
        ge = cnt >= kp1
        return jnp.where(ge, lo, mid + 1), jnp.where(ge, mid, hi)

    _, t = lax.fori_loop(0, 31, bs_body, (lo0, hi0))

    lt = (bits < t).astype(jnp.bfloat16)
    eq = (bits == t).astype(jnp.bfloat16)
    mn = jnp.min(bits, axis=1, keepdims=True)
    eqm = (bits == mn).astype(jnp.bfloat16)
    n_lt = lax.dot_general(lt, ones_colA, (((1,), (0,)), ((), ())),
                           preferred_element_type=jnp.float32)

    ii = lax.broadcasted_iota(jnp.int32, (_A, _A), 0)
    jj = lax.broadcasted_iota(jnp.int32, (_A, _A), 1)
    triu = (ii <= jj).astype(jnp.bfloat16)
    stacked = jnp.concatenate([eq, eqm], axis=0)
    pref = lax.dot_general(stacked, triu, (((1,), (0,)), ((), ())),
                           preferred_element_type=jnp.float32)
    pref_eq = pref[0:_N, :]
    pref_mn = pref[_N:, :]

    r = kp1 - n_lt
    sel = lt + eq * (pref_eq <= r).astype(jnp.bfloat16)
    first_min = eqm * (pref_mn == 1.0).astype(jnp.bfloat16)
    M = (sel - first_min).astype(jnp.float32)

    ones_colAf = jnp.ones((_A, 1), jnp.float32)
    dinv_parts = []
    for e in range(_C):
        Me = M[e * _A:(e + 1) * _A, :]
        deg_e = lax.dot_general(Me, ones_colAf, (((0,), (0,)), ((), ())),
                                preferred_element_type=jnp.float32) + 2.0
        dinv_parts.append(1.0 / jnp.sqrt(deg_e))
    dinv = jnp.concatenate(dinv_parts, axis=0)

    def conv(xin, W, b):
        xt = lax.dot_general(xin, W, (((1,), (1,)), ((), ())),
                             preferred_element_type=jnp.float32,
                             precision=_H3)
        u = dinv * xt
        zs = []
        for e in range(_C):
            Me = M[e * _A:(e + 1) * _A, :]
            ue = u[e * _A:(e + 1) * _A, :]
            zs.append(lax.dot_general(Me, ue, (((0,), (0,)), ((), ())),
                                      preferred_element_type=jnp.float32,
                                      precision=_H3))
        z = jnp.concatenate(zs, axis=0)
        return dinv * z + 2.0 * dinv * u + b

    h1 = jnp.tanh(conv(X, W1_ref[...], b1_ref[...]))
    h2 = jnp.tanh(conv(h1, W2_ref[...], b2_ref[...]))
    out = lax.dot_general(h2, Wo_ref[...], (((1,), (1,)), ((), ())),
                          preferred_element_type=jnp.float32,
                          precision=_H3) + bo_ref[...]
    out_ref[...] = out.reshape(_C, _A, _OUT)


def _build(interpret=False):
    return pl.pallas_call(
        _env_kernel,
        grid=(_B // _C,),
        in_specs=[
            pl.BlockSpec((_C, _A, _D), lambda b: (b, 0, 0)),
            pl.BlockSpec((_H, _D), lambda b: (0, 0)),
            pl.BlockSpec((1, _H), lambda b: (0, 0)),
            pl.BlockSpec((_H, _H), lambda b: (0, 0)),
            pl.BlockSpec((1, _H), lambda b: (0, 0)),
            pl.BlockSpec((_OUT, _H), lambda b: (0, 0)),
            pl.BlockSpec((1, _OUT), lambda b: (0, 0)),
        ],
        out_specs=pl.BlockSpec((_C, _A, _OUT), lambda b: (b, 0, 0)),
        out_shape=jax.ShapeDtypeStruct((_B, _A, _OUT), jnp.float32),
        interpret=interpret,
    )


@jax.jit
def kernel(agent_observations, W1, b1, W2, b2, Wo, bo):
    obs = agent_observations.astype(jnp.float32)
    return _build()(obs, W1, b1.reshape(1, _H), W2, b2.reshape(1, _H),
                    Wo, bo.reshape(1, _OUT))


# speedup vs baseline: 99.3304x; 1.9603x over previous
import jax
import jax.numpy as jnp
from jax import lax
from jax.experimental import pallas as pl

_B, _A, _D, _H, _OUT, _K = 50, 200, 128, 128, 32, 32
_C = 10
_N = _C * _A

_HI = lax.Precision.HIGHEST


def _env_kernel(obs_ref, W1_ref, b1_ref, W2_ref, b2_ref, Wo_ref, bo_ref, out_ref):
    obs = obs_ref[...]
    X = obs.reshape(_N, _D)
    f32 = jnp.float32
    bf16 = jnp.bfloat16

    px_row = obs[:, :, 0]
    py_row = obs[:, :, 1]
    eyeC = (lax.broadcasted_iota(jnp.int32, (_C, _C), 0) ==
            lax.broadcasted_iota(jnp.int32, (_C, _C), 1)).astype(f32)
    pxT = lax.dot_general(px_row, eyeC, (((0,), (0,)), ((), ())),
                          preferred_element_type=f32, precision=_HI)
    pyT = lax.dot_general(py_row, eyeC, (((0,), (0,)), ((), ())),
                          preferred_element_type=f32, precision=_HI)
    gE = lax.broadcasted_iota(jnp.int32, (_C, _N), 1) // _A
    ET = (gE == lax.broadcasted_iota(jnp.int32, (_C, _N), 0)).astype(f32)
    PCx = lax.dot_general(pxT, ET, (((1,), (0,)), ((), ())),
                          preferred_element_type=f32, precision=_HI)
    PCy = lax.dot_general(pyT, ET, (((1,), (0,)), ((), ())),
                          preferred_element_type=f32, precision=_HI)
    jmod = lax.rem(lax.broadcasted_iota(jnp.int32, (_A, _N), 1), _A)
    JSel = (jmod == lax.broadcasted_iota(jnp.int32, (_A, _N), 0)).astype(f32)
    ones_rowAf = jnp.ones((1, _A), f32)
    px_flat = lax.dot_general(ones_rowAf, PCx * JSel, (((1,), (0,)), ((), ())),
                              preferred_element_type=f32, precision=_HI)
    py_flat = lax.dot_general(ones_rowAf, PCy * JSel, (((1,), (0,)), ((), ())),
                              preferred_element_type=f32, precision=_HI)
    dxT = px_flat - PCx
    dyT = py_flat - PCy
    dT = jnp.sqrt(dxT * dxT + dyT * dyT)

    bitsT = lax.bitcast_convert_type(dT, jnp.int32)
    kp1 = float(_K + 1)
    ones_rowA = jnp.ones((1, _A), bf16)
    lo0 = jnp.zeros((1, _N), jnp.int32)
    hi0 = jnp.full((1, _N), 0x7F800000, jnp.int32)

    def bs_body(_, carry):
        lo, hi = carry
        mid = lo + lax.div(hi - lo, 2)
        le = (bitsT <= mid).astype(bf16)
        cnt = lax.dot_general(ones_rowA, le, (((1,), (0,)), ((), ())),
                              preferred_element_type=f32)
        ge = cnt >= kp1
        return jnp.where(ge, lo, mid + 1), jnp.where(ge, mid, hi)

    _, t = lax.fori_loop(0, 31, bs_body, (lo0, hi0))

    ltT = (bitsT < t).astype(bf16)
    eqT = (bitsT == t).astype(bf16)
    mn = jnp.min(bitsT, axis=0, keepdims=True)
    eqmT = (bitsT == mn).astype(bf16)
    n_lt = lax.dot_general(ones_rowA, ltT, (((1,), (0,)), ((), ())),
                           preferred_element_type=f32)

    Ltri = (lax.broadcasted_iota(jnp.int32, (_A, _A), 1) <=
            lax.broadcasted_iota(jnp.int32, (_A, _A), 0)).astype(bf16)
    pref_eq = lax.dot_general(Ltri, eqT, (((1,), (0,)), ((), ())),
                              preferred_element_type=f32)
    pref_mn = lax.dot_general(Ltri, eqmT, (((1,), (0,)), ((), ())),
                              preferred_element_type=f32)

    r = kp1 - n_lt
    selT = ltT + eqT * (pref_eq <= r).astype(bf16)
    firstT = eqmT * (pref_mn == 1.0).astype(bf16)
    MT = selT - firstT

    eyeA = (lax.broadcasted_iota(jnp.int32, (_A, _A), 0) ==
            lax.broadcasted_iota(jnp.int32, (_A, _A), 1)).astype(bf16)
    M = lax.dot_general(MT, eyeA, (((0,), (0,)), ((), ())),
                        preferred_element_type=f32)

    ones_colAf = jnp.ones((_A, 1), f32)
    dinv_parts = []
    for e in range(_C):
        Me = M[e * _A:(e + 1) * _A, :]
        deg_e = lax.dot_general(Me, ones_colAf, (((0,), (0,)), ((), ())),
                                preferred_element_type=f32) + 2.0
        dinv_parts.append(1.0 / jnp.sqrt(deg_e))
    dinv = jnp.concatenate(dinv_parts, axis=0)

    def conv(xin, W, b):
        xt = lax.dot_general(xin, W, (((1,), (1,)), ((), ())),
                             preferred_element_type=f32)
        u = dinv * xt
        zs = []
        for e in range(_C):
            Me = M[e * _A:(e + 1) * _A, :]
            ue = u[e * _A:(e + 1) * _A, :]
            zs.append(lax.dot_general(Me, ue, (((0,), (0,)), ((), ())),
                                      preferred_element_type=f32))
        z = jnp.concatenate(zs, axis=0)
        return dinv * z + 2.0 * dinv * u + b

    h1 = jnp.tanh(conv(X, W1_ref[...], b1_ref[...]))
    h2 = jnp.tanh(conv(h1, W2_ref[...], b2_ref[...]))
    out = lax.dot_general(h2, Wo_ref[...], (((1,), (1,)), ((), ())),
                          preferred_element_type=f32) + bo_ref[...]
    out_ref[...] = out.reshape(_C, _A, _OUT)


def _build(interpret=False):
    return pl.pallas_call(
        _env_kernel,
        grid=(_B // _C,),
        in_specs=[
            pl.BlockSpec((_C, _A, _D), lambda b: (b, 0, 0)),
            pl.BlockSpec((_H, _D), lambda b: (0, 0)),
            pl.BlockSpec((1, _H), lambda b: (0, 0)),
            pl.BlockSpec((_H, _H), lambda b: (0, 0)),
            pl.BlockSpec((1, _H), lambda b: (0, 0)),
            pl.BlockSpec((_OUT, _H), lambda b: (0, 0)),
            pl.BlockSpec((1, _OUT), lambda b: (0, 0)),
        ],
        out_specs=pl.BlockSpec((_C, _A, _OUT), lambda b: (b, 0, 0)),
        out_shape=jax.ShapeDtypeStruct((_B, _A, _OUT), jnp.float32),
        interpret=interpret,
    )


@jax.jit
def kernel(agent_observations, W1, b1, W2, b2, Wo, bo):
    obs = agent_observations.astype(jnp.float32)
    return _build()(obs, W1, b1.reshape(1, _H), W2, b2.reshape(1, _H),
                    Wo, bo.reshape(1, _OUT))


# speedup vs baseline: 113.5278x; 1.1429x over previous
import jax
import jax.numpy as jnp
from jax import lax
from jax.experimental import pallas as pl

_B, _A, _D, _H, _OUT, _K = 50, 200, 128, 128, 32, 32
_C = 25
_N = _C * _A

_HI = lax.Precision.HIGHEST


def _env_kernel(obs_ref, W1_ref, b1_ref, W2_ref, b2_ref, Wo_ref, bo_ref, out_ref):
    obs = obs_ref[...]
    X = obs.reshape(_N, _D)
    f32 = jnp.float32
    bf16 = jnp.bfloat16

    px_row = obs[:, :, 0]
    py_row = obs[:, :, 1]
    eyeC = (lax.broadcasted_iota(jnp.int32, (_C, _C), 0) ==
            lax.broadcasted_iota(jnp.int32, (_C, _C), 1)).astype(f32)
    pxT = lax.dot_general(px_row, eyeC, (((0,), (0,)), ((), ())),
                          preferred_element_type=f32, precision=_HI)
    pyT = lax.dot_general(py_row, eyeC, (((0,), (0,)), ((), ())),
                          preferred_element_type=f32, precision=_HI)
    gE = lax.broadcasted_iota(jnp.int32, (_C, _N), 1) // _A
    ET = (gE == lax.broadcasted_iota(jnp.int32, (_C, _N), 0)).astype(f32)
    PCx = lax.dot_general(pxT, ET, (((1,), (0,)), ((), ())),
                          preferred_element_type=f32, precision=_HI)
    PCy = lax.dot_general(pyT, ET, (((1,), (0,)), ((), ())),
                          preferred_element_type=f32, precision=_HI)
    jmod = lax.rem(lax.broadcasted_iota(jnp.int32, (_A, _N), 1), _A)
    JSel = (jmod == lax.broadcasted_iota(jnp.int32, (_A, _N), 0)).astype(f32)
    ones_rowAf = jnp.ones((1, _A), f32)
    px_flat = lax.dot_general(ones_rowAf, PCx * JSel, (((1,), (0,)), ((), ())),
                              preferred_element_type=f32, precision=_HI)
    py_flat = lax.dot_general(ones_rowAf, PCy * JSel, (((1,), (0,)), ((), ())),
                              preferred_element_type=f32, precision=_HI)
    dxT = px_flat - PCx
    dyT = py_flat - PCy
    dT = jnp.sqrt(dxT * dxT + dyT * dyT)

    bitsT = lax.bitcast_convert_type(dT, jnp.int32)
    kp1 = float(_K + 1)
    ones_rowA = jnp.ones((1, _A), bf16)
    lo0 = jnp.zeros((1, _N), jnp.int32)
    hi0 = jnp.full((1, _N), 0x7F800000, jnp.int32)

    def bs_body(_, carry):
        lo, hi = carry
        mid = lo + lax.div(hi - lo, 2)
        le = (bitsT <= mid).astype(bf16)
        cnt = lax.dot_general(ones_rowA, le, (((1,), (0,)), ((), ())),
                              preferred_element_type=f32)
        ge = cnt >= kp1
        return jnp.where(ge, lo, mid + 1), jnp.where(ge, mid, hi)

    _, t = lax.fori_loop(0, 31, bs_body, (lo0, hi0))

    ltT = (bitsT < t).astype(bf16)
    eqT = (bitsT == t).astype(bf16)
    mn = jnp.min(bitsT, axis=0, keepdims=True)
    eqmT = (bitsT == mn).astype(bf16)
    n_lt = lax.dot_general(ones_rowA, ltT, (((1,), (0,)), ((), ())),
                           preferred_element_type=f32)

    Ltri = (lax.broadcasted_iota(jnp.int32, (_A, _A), 1) <=
            lax.broadcasted_iota(jnp.int32, (_A, _A), 0)).astype(bf16)
    pref_eq = lax.dot_general(Ltri, eqT, (((1,), (0,)), ((), ())),
                              preferred_element_type=f32)
    pref_mn = lax.dot_general(Ltri, eqmT, (((1,), (0,)), ((), ())),
                              preferred_element_type=f32)

    r = kp1 - n_lt
    selT = ltT + eqT * (pref_eq <= r).astype(bf16)
    firstT = eqmT * (pref_mn == 1.0).astype(bf16)
    MT = selT - firstT

    eyeA = (lax.broadcasted_iota(jnp.int32, (_A, _A), 0) ==
            lax.broadcasted_iota(jnp.int32, (_A, _A), 1)).astype(bf16)
    M = lax.dot_general(MT, eyeA, (((0,), (0,)), ((), ())),
                        preferred_element_type=f32)

    ones_colAf = jnp.ones((_A, 1), f32)
    dinv_parts = []
    for e in range(_C):
        Me = M[e * _A:(e + 1) * _A, :]
        deg_e = lax.dot_general(Me, ones_colAf, (((0,), (0,)), ((), ())),
                                preferred_element_type=f32) + 2.0
        dinv_parts.append(1.0 / jnp.sqrt(deg_e))
    dinv = jnp.concatenate(dinv_parts, axis=0)

    def conv(xin, W, b):
        xt = lax.dot_general(xin, W, (((1,), (1,)), ((), ())),
                             preferred_element_type=f32)
        u = dinv * xt
        zs = []
        for e in range(_C):
            Me = M[e * _A:(e + 1) * _A, :]
            ue = u[e * _A:(e + 1) * _A, :]
            zs.append(lax.dot_general(Me, ue, (((0,), (0,)), ((), ())),
                                      preferred_element_type=f32))
        z = jnp.concatenate(zs, axis=0)
        return dinv * z + 2.0 * dinv * u + b

    h1 = jnp.tanh(conv(X, W1_ref[...], b1_ref[...]))
    h2 = jnp.tanh(conv(h1, W2_ref[...], b2_ref[...]))
    out = lax.dot_general(h2, Wo_ref[...], (((1,), (1,)), ((), ())),
                          preferred_element_type=f32) + bo_ref[...]
    out_ref[...] = out.reshape(_C, _A, _OUT)


def _build(interpret=False):
    return pl.pallas_call(
        _env_kernel,
        grid=(_B // _C,),
        in_specs=[
            pl.BlockSpec((_C, _A, _D), lambda b: (b, 0, 0)),
            pl.BlockSpec((_H, _D), lambda b: (0, 0)),
            pl.BlockSpec((1, _H), lambda b: (0, 0)),
            pl.BlockSpec((_H, _H), lambda b: (0, 0)),
            pl.BlockSpec((1, _H), lambda b: (0, 0)),
            pl.BlockSpec((_OUT, _H), lambda b: (0, 0)),
            pl.BlockSpec((1, _OUT), lambda b: (0, 0)),
        ],
        out_specs=pl.BlockSpec((_C, _A, _OUT), lambda b: (b, 0, 0)),
        out_shape=jax.ShapeDtypeStruct((_B, _A, _OUT), jnp.float32),
        interpret=interpret,
    )


@jax.jit
def kernel(agent_observations, W1, b1, W2, b2, Wo, bo):
    obs = agent_observations.astype(jnp.float32)
    return _build()(obs, W1, b1.reshape(1, _H), W2, b2.reshape(1, _H),
                    Wo, bo.reshape(1, _OUT))
